# baseline (speedup 1.0000x reference)
import functools

import jax
import jax.numpy as jnp
from jax import lax
from jax.experimental import pallas as pl
from jax.experimental.pallas import tpu as pltpu
from jax.experimental.pallas import tpu_sc as plsc

D_MODEL = 64
NUM_CORES = 2
NUM_SUBCORES = 16
NW = NUM_CORES * NUM_SUBCORES
LANES = 16


def _sc_embed_body(seq, bw, lut_hbm, xt_hbm, out_hbm,
                   lut_v, raw0, raw1, blk0, blk1,
                   lsem, isem0, isem1, osem0, osem1):
    wid = lax.axis_index("s") * NUM_CORES + lax.axis_index("c")
    b0 = wid * bw

    raw = (raw0, raw1)
    blk = (blk0, blk1)
    isem = (isem0, isem1)
    osem = (osem0, osem1)

    pltpu.async_copy(lut_hbm, lut_v, lsem).wait()

    def idx_copy(s, p):
        return pltpu.make_async_copy(
            xt_hbm.at[s, pl.ds(b0, bw)], raw[p], isem[p])

    def out_copy(s, p):
        return pltpu.make_async_copy(
            blk[p], out_hbm.at[s, :, pl.ds(b0, bw)], osem[p])

    idx_copy(0, 0).start()
    idx_copy(1, 1).start()

    def expand(p):
        def kbody(k, carry):
            t_vec = raw[p][pl.ds(LANES * k, LANES)]
            base = t_vec * D_MODEL
            for d in range(D_MODEL):
                blk[p][d, pl.ds(LANES * k, LANES)] = plsc.load_gather(
                    lut_v, [base + d])
            return carry
        lax.fori_loop(0, bw // LANES, kbody, 0)

    def body(j, carry):
        for p in (0, 1):
            s = 2 * j + p
            idx_copy(s, p).wait()
            @pl.when(j >= 1)
            def _():
                out_copy(s - 2, p).wait()
            expand(p)
            @pl.when(j < seq // 2 - 1)
            def _():
                idx_copy(s + 2, p).start()
            out_copy(s, p).start()
        return carry

    lax.fori_loop(0, seq // 2, body, 0)
    out_copy(seq - 2, 0).wait()
    out_copy(seq - 1, 1).wait()


def kernel(x, lut):
    batch, seq = x.shape
    assert batch % NW == 0 and seq % 2 == 0
    bw = batch // NW
    xt = jnp.transpose(x).astype(jnp.int32)
    lut_flat = lut.reshape(-1)

    mesh = plsc.VectorSubcoreMesh(core_axis_name="c", subcore_axis_name="s")
    run = pl.kernel(
        functools.partial(_sc_embed_body, seq, bw),
        mesh=mesh,
        compiler_params=pltpu.CompilerParams(needs_layout_passes=False),
        out_type=jax.ShapeDtypeStruct((seq, D_MODEL, batch), jnp.float32),
        scratch_types=[
            pltpu.VMEM((lut_flat.shape[0],), jnp.float32),
            pltpu.VMEM((bw,), jnp.int32),
            pltpu.VMEM((bw,), jnp.int32),
            pltpu.VMEM((D_MODEL, bw), jnp.float32),
            pltpu.VMEM((D_MODEL, bw), jnp.float32),
            pltpu.SemaphoreType.DMA,
            pltpu.SemaphoreType.DMA,
            pltpu.SemaphoreType.DMA,
            pltpu.SemaphoreType.DMA,
            pltpu.SemaphoreType.DMA,
        ],
    )
    out = run(lut_flat, xt)
    return jnp.transpose(out, (2, 0, 1))


# speedup vs baseline: 1.8880x; 1.8880x over previous
import functools

import jax
import jax.numpy as jnp
from jax import lax
from jax.experimental import pallas as pl
from jax.experimental.pallas import tpu as pltpu
from jax.experimental.pallas import tpu_sc as plsc

D_MODEL = 64
NUM_CORES = 2
NUM_SUBCORES = 16
NW = NUM_CORES * NUM_SUBCORES
LANES = 16


def _sc_embed_body(seq, bw, lut_hbm, xt_hbm, out_hbm,
                   lut_v, raw0, raw1, blk0, blk1,
                   lsem, isem0, isem1, osem0, osem1):
    wid = lax.axis_index("s") * NUM_CORES + lax.axis_index("c")
    b0 = wid * bw

    raw = (raw0, raw1)
    blk = (blk0, blk1)
    isem = (isem0, isem1)
    osem = (osem0, osem1)

    pltpu.async_copy(lut_hbm, lut_v, lsem).wait()

    def idx_copy(s, p):
        return pltpu.make_async_copy(
            xt_hbm.at[s, pl.ds(b0, bw)], raw[p], isem[p])

    def out_copy(s, p):
        return pltpu.make_async_copy(
            blk[p], out_hbm.at[s, :, pl.ds(b0, bw)], osem[p])

    idx_copy(0, 0).start()
    idx_copy(1, 1).start()

    def expand(p):
        @plsc.parallel_loop(0, bw // LANES, unroll=2)
        def kbody(k):
            t_vec = raw[p][pl.ds(LANES * k, LANES)]
            base = t_vec * D_MODEL
            @plsc.parallel_loop(0, D_MODEL, unroll=8)
            def dbody(d):
                blk[p][d, pl.ds(LANES * k, LANES)] = plsc.load_gather(
                    lut_v, [base + d])

    def body(j, carry):
        for p in (0, 1):
            s = 2 * j + p
            idx_copy(s, p).wait()
            @pl.when(j >= 1)
            def _():
                out_copy(s - 2, p).wait()
            expand(p)
            @pl.when(j < seq // 2 - 1)
            def _():
                idx_copy(s + 2, p).start()
            out_copy(s, p).start()
        return carry

    lax.fori_loop(0, seq // 2, body, 0)
    out_copy(seq - 2, 0).wait()
    out_copy(seq - 1, 1).wait()


def kernel(x, lut):
    batch, seq = x.shape
    assert batch % NW == 0 and seq % 2 == 0
    bw = batch // NW
    xt = jnp.transpose(x).astype(jnp.int32)
    lut_flat = lut.reshape(-1)

    mesh = plsc.VectorSubcoreMesh(core_axis_name="c", subcore_axis_name="s")
    run = pl.kernel(
        functools.partial(_sc_embed_body, seq, bw),
        mesh=mesh,
        compiler_params=pltpu.CompilerParams(needs_layout_passes=False),
        out_type=jax.ShapeDtypeStruct((seq, D_MODEL, batch), jnp.float32),
        scratch_types=[
            pltpu.VMEM((lut_flat.shape[0],), jnp.float32),
            pltpu.VMEM((bw,), jnp.int32),
            pltpu.VMEM((bw,), jnp.int32),
            pltpu.VMEM((D_MODEL, bw), jnp.float32),
            pltpu.VMEM((D_MODEL, bw), jnp.float32),
            pltpu.SemaphoreType.DMA,
            pltpu.SemaphoreType.DMA,
            pltpu.SemaphoreType.DMA,
            pltpu.SemaphoreType.DMA,
            pltpu.SemaphoreType.DMA,
        ],
    )
    out = run(lut_flat, xt)
    return jnp.transpose(out, (2, 0, 1))
